# baseline (device time: 176068 ns/iter reference)
import jax
import jax.numpy as jnp
from jax import lax
from jax.experimental import pallas as pl
from jax.experimental.pallas import tpu as pltpu

N_DEV = 4


def kernel(x, w_mat, scale_x, scale_w):
    m_global, k_shard = x.shape
    _, n = w_mat.shape
    m_per = m_global // N_DEV

    x = x.astype(jnp.bfloat16)
    w_mat = w_mat.astype(jnp.bfloat16)

    def body(x_ref, w_ref, sx_ref, sw_ref, out_ref,
             send_buf, recv_buf, send_sems, recv_sems):
        my = lax.axis_index("i")
        left = lax.rem(my + N_DEV - 1, N_DEV)
        right = lax.rem(my + 1, N_DEV)

        barrier_sem = pltpu.get_barrier_semaphore()
        for nbr in (left, right):
            pl.semaphore_signal(
                barrier_sem, inc=1,
                device_id=(nbr,), device_id_type=pl.DeviceIdType.MESH,
            )
        pl.semaphore_wait(barrier_sem, 2)

        def partial(c):
            xs = x_ref[pl.ds(c * m_per, m_per), :]
            return jnp.dot(xs, w_ref[...], preferred_element_type=jnp.float32)

        rdmas = []
        for s in range(N_DEV - 1):
            c = lax.rem(my - (s + 1) + N_DEV, N_DEV)
            val = partial(c)
            if s > 0:
                rdmas[s - 1].wait_recv()
                val = val + recv_buf[s - 1].astype(jnp.float32)
            send_buf[s] = val.astype(jnp.bfloat16)
            rdma = pltpu.make_async_remote_copy(
                src_ref=send_buf.at[s],
                dst_ref=recv_buf.at[s],
                send_sem=send_sems.at[s],
                recv_sem=recv_sems.at[s],
                device_id=(right,),
                device_id_type=pl.DeviceIdType.MESH,
            )
            rdma.start()
            rdmas.append(rdma)

        p_own = partial(my)
        rdmas[N_DEV - 2].wait_recv()
        scale = sx_ref[0] * sw_ref[0]
        out_ref[...] = (
            p_own + recv_buf[N_DEV - 2].astype(jnp.float32)
        ) * scale
        for r in rdmas:
            r.wait_send()

    return pl.pallas_call(
        body,
        out_shape=jax.ShapeDtypeStruct((m_per, n), jnp.float32),
        in_specs=[
            pl.BlockSpec(memory_space=pltpu.VMEM),
            pl.BlockSpec(memory_space=pltpu.VMEM),
            pl.BlockSpec(memory_space=pltpu.SMEM),
            pl.BlockSpec(memory_space=pltpu.SMEM),
        ],
        out_specs=pl.BlockSpec(memory_space=pltpu.VMEM),
        scratch_shapes=[
            pltpu.VMEM((N_DEV - 1, m_per, n), jnp.bfloat16),
            pltpu.VMEM((N_DEV - 1, m_per, n), jnp.bfloat16),
            pltpu.SemaphoreType.DMA((N_DEV - 1,)),
            pltpu.SemaphoreType.DMA((N_DEV - 1,)),
        ],
        compiler_params=pltpu.CompilerParams(collective_id=0),
    )(x, w_mat, scale_x, scale_w)


# device time: 157961 ns/iter; 1.1146x vs baseline; 1.1146x over previous
import os

import jax
import jax.numpy as jnp
from jax import lax
from jax.experimental import pallas as pl
from jax.experimental.pallas import tpu as pltpu

N_DEV = 4
try:
    _MODE = open(os.path.join(os.path.dirname(__file__),
                              "kernel_mode.txt")).read().strip()
except OSError:
    _MODE = "full"


def kernel(x, w_mat, scale_x, scale_w):
    m_global, k_shard = x.shape
    _, n = w_mat.shape
    m_per = m_global // N_DEV

    x = x.astype(jnp.bfloat16)
    w_mat = w_mat.astype(jnp.bfloat16)

    def body(x_ref, w_ref, sx_ref, sw_ref, out_ref,
             send_buf, recv_buf, send_sems, recv_sems):
        my = lax.axis_index("i")
        left = lax.rem(my + N_DEV - 1, N_DEV)
        right = lax.rem(my + 1, N_DEV)

        barrier_sem = pltpu.get_barrier_semaphore()
        for nbr in (left, right):
            pl.semaphore_signal(
                barrier_sem, inc=1,
                device_id=(nbr,), device_id_type=pl.DeviceIdType.MESH,
            )
        pl.semaphore_wait(barrier_sem, 2)

        def partial(c):
            if _MODE == "nocompute":
                return jnp.zeros((m_per, n), jnp.float32)
            xs = x_ref[pl.ds(c * m_per, m_per), :]
            return jnp.dot(xs, w_ref[...], preferred_element_type=jnp.float32)

        if _MODE == "nocomm":
            acc = partial(my)
            for s in range(N_DEV - 1):
                c = lax.rem(my - (s + 1) + N_DEV, N_DEV)
                send_buf[s] = partial(c).astype(jnp.bfloat16)
                acc = acc + send_buf[s].astype(jnp.float32)
            out_ref[...] = acc * (sx_ref[0] * sw_ref[0])
            return

        rdmas = []
        for s in range(N_DEV - 1):
            c = lax.rem(my - (s + 1) + N_DEV, N_DEV)
            val = partial(c)
            if s > 0:
                rdmas[s - 1].wait_recv()
                val = val + recv_buf[s - 1].astype(jnp.float32)
            send_buf[s] = val.astype(jnp.bfloat16)
            rdma = pltpu.make_async_remote_copy(
                src_ref=send_buf.at[s],
                dst_ref=recv_buf.at[s],
                send_sem=send_sems.at[s],
                recv_sem=recv_sems.at[s],
                device_id=(right,),
                device_id_type=pl.DeviceIdType.MESH,
            )
            rdma.start()
            rdmas.append(rdma)

        p_own = partial(my)
        rdmas[N_DEV - 2].wait_recv()
        scale = sx_ref[0] * sw_ref[0]
        out_ref[...] = (
            p_own + recv_buf[N_DEV - 2].astype(jnp.float32)
        ) * scale
        for r in rdmas:
            r.wait_send()

    return pl.pallas_call(
        body,
        out_shape=jax.ShapeDtypeStruct((m_per, n), jnp.float32),
        in_specs=[
            pl.BlockSpec(memory_space=pltpu.VMEM),
            pl.BlockSpec(memory_space=pltpu.VMEM),
            pl.BlockSpec(memory_space=pltpu.SMEM),
            pl.BlockSpec(memory_space=pltpu.SMEM),
        ],
        out_specs=pl.BlockSpec(memory_space=pltpu.VMEM),
        scratch_shapes=[
            pltpu.VMEM((N_DEV - 1, m_per, n), jnp.bfloat16),
            pltpu.VMEM((N_DEV - 1, m_per, n), jnp.bfloat16),
            pltpu.SemaphoreType.DMA((N_DEV - 1,)),
            pltpu.SemaphoreType.DMA((N_DEV - 1,)),
        ],
        compiler_params=pltpu.CompilerParams(collective_id=0),
    )(x, w_mat, scale_x, scale_w)


# device time: 156102 ns/iter; 1.1279x vs baseline; 1.0119x over previous
import os

import jax
import jax.numpy as jnp
from jax import lax
from jax.experimental import pallas as pl
from jax.experimental.pallas import tpu as pltpu

N_DEV = 4
try:
    _MODE = open(os.path.join(os.path.dirname(__file__),
                              "kernel_mode.txt")).read().strip()
except OSError:
    _MODE = "full"


def kernel(x, w_mat, scale_x, scale_w):
    m_global, k_shard = x.shape
    _, n = w_mat.shape
    m_per = m_global // N_DEV

    x = x.astype(jnp.bfloat16)
    w_mat = w_mat.astype(jnp.bfloat16)

    def body(x_ref, w_ref, sx_ref, sw_ref, out_ref,
             send_buf, recv_buf, send_sems, recv_sems):
        my = lax.axis_index("i")
        left = lax.rem(my + N_DEV - 1, N_DEV)
        right = lax.rem(my + 1, N_DEV)

        barrier_sem = pltpu.get_barrier_semaphore()
        for nbr in (left, right):
            pl.semaphore_signal(
                barrier_sem, inc=1,
                device_id=(nbr,), device_id_type=pl.DeviceIdType.MESH,
            )
        pl.semaphore_wait(barrier_sem, 2)

        def partial(c):
            if _MODE == "nocompute":
                return jnp.zeros((m_per, n), jnp.float32)
            xs = x_ref[pl.ds(c * m_per, m_per), :]
            return jnp.dot(xs, w_ref[...], preferred_element_type=jnp.float32)

        if _MODE == "split":
            h = m_per // 2
            for s in range(N_DEV - 1):
                rs = []
                for j in range(2):
                    r = pltpu.make_async_remote_copy(
                        src_ref=send_buf.at[s, pl.ds(j * h, h)],
                        dst_ref=recv_buf.at[s, pl.ds(j * h, h)],
                        send_sem=send_sems.at[s, j],
                        recv_sem=recv_sems.at[s, j],
                        device_id=(right,),
                        device_id_type=pl.DeviceIdType.MESH,
                    )
                    r.start()
                    rs.append(r)
                for r in rs:
                    r.wait()
            out_ref[...] = recv_buf[N_DEV - 2].astype(jnp.float32)
            return

        if _MODE == "nocomm":
            acc = partial(my)
            for s in range(N_DEV - 1):
                c = lax.rem(my - (s + 1) + N_DEV, N_DEV)
                send_buf[s] = partial(c).astype(jnp.bfloat16)
                acc = acc + send_buf[s].astype(jnp.float32)
            out_ref[...] = acc * (sx_ref[0] * sw_ref[0])
            return

        rdmas = []
        for s in range(N_DEV - 1):
            c = lax.rem(my - (s + 1) + N_DEV, N_DEV)
            val = partial(c)
            if s > 0:
                rdmas[s - 1].wait_recv()
                val = val + recv_buf[s - 1].astype(jnp.float32)
            send_buf[s] = val.astype(jnp.bfloat16)
            rdma = pltpu.make_async_remote_copy(
                src_ref=send_buf.at[s],
                dst_ref=recv_buf.at[s],
                send_sem=send_sems.at[s, 0],
                recv_sem=recv_sems.at[s, 0],
                device_id=(right,),
                device_id_type=pl.DeviceIdType.MESH,
            )
            rdma.start()
            rdmas.append(rdma)

        p_own = partial(my)
        rdmas[N_DEV - 2].wait_recv()
        scale = sx_ref[0] * sw_ref[0]
        out_ref[...] = (
            p_own + recv_buf[N_DEV - 2].astype(jnp.float32)
        ) * scale
        for r in rdmas:
            r.wait_send()

    return pl.pallas_call(
        body,
        out_shape=jax.ShapeDtypeStruct((m_per, n), jnp.float32),
        in_specs=[
            pl.BlockSpec(memory_space=pltpu.VMEM),
            pl.BlockSpec(memory_space=pltpu.VMEM),
            pl.BlockSpec(memory_space=pltpu.SMEM),
            pl.BlockSpec(memory_space=pltpu.SMEM),
        ],
        out_specs=pl.BlockSpec(memory_space=pltpu.VMEM),
        scratch_shapes=[
            pltpu.VMEM((N_DEV - 1, m_per, n), jnp.bfloat16),
            pltpu.VMEM((N_DEV - 1, m_per, n), jnp.bfloat16),
            pltpu.SemaphoreType.DMA((N_DEV - 1, 2)),
            pltpu.SemaphoreType.DMA((N_DEV - 1, 2)),
        ],
        compiler_params=pltpu.CompilerParams(collective_id=0),
    )(x, w_mat, scale_x, scale_w)


# device time: 89877 ns/iter; 1.9590x vs baseline; 1.7368x over previous
import os

import jax
import jax.numpy as jnp
from jax import lax
from jax.experimental import pallas as pl
from jax.experimental.pallas import tpu as pltpu

N_DEV = 4
try:
    _MODE = open(os.path.join(os.path.dirname(__file__),
                              "kernel_mode.txt")).read().strip()
except OSError:
    _MODE = "full"


def kernel(x, w_mat, scale_x, scale_w):
    m_global, k_shard = x.shape
    _, n = w_mat.shape
    m_per = m_global // N_DEV

    x = x.astype(jnp.bfloat16)
    w_mat = w_mat.astype(jnp.bfloat16)

    def body(x_ref, w_ref, sx_ref, sw_ref, out_ref,
             send_buf, recv_buf, recv2_buf, send_sems, recv_sems):
        my = lax.axis_index("i")
        left = lax.rem(my + N_DEV - 1, N_DEV)
        right = lax.rem(my + 1, N_DEV)

        barrier_sem = pltpu.get_barrier_semaphore()
        for nbr in (left, right):
            pl.semaphore_signal(
                barrier_sem, inc=1,
                device_id=(nbr,), device_id_type=pl.DeviceIdType.MESH,
            )
        pl.semaphore_wait(barrier_sem, 2)

        def partial(c):
            if _MODE == "nocompute":
                return jnp.zeros((m_per, n), jnp.float32)
            xs = x_ref[pl.ds(c * m_per, m_per), :]
            return jnp.dot(xs, w_ref[...], preferred_element_type=jnp.float32)

        if _MODE == "half":
            h = m_per // 2
            for s in range(N_DEV - 1):
                r = pltpu.make_async_remote_copy(
                    src_ref=send_buf.at[s, pl.ds(0, h)],
                    dst_ref=recv_buf.at[s, pl.ds(0, h)],
                    send_sem=send_sems.at[s, 0],
                    recv_sem=recv_sems.at[s, 0],
                    device_id=(right,),
                    device_id_type=pl.DeviceIdType.MESH,
                )
                r.start()
                r.wait()
            out_ref[...] = recv_buf[N_DEV - 2].astype(jnp.float32)
            return

        if _MODE == "bidir":
            for s in range(N_DEV - 1):
                rr = pltpu.make_async_remote_copy(
                    src_ref=send_buf.at[s],
                    dst_ref=recv_buf.at[s],
                    send_sem=send_sems.at[s, 0],
                    recv_sem=recv_sems.at[s, 0],
                    device_id=(right,),
                    device_id_type=pl.DeviceIdType.MESH,
                )
                rl = pltpu.make_async_remote_copy(
                    src_ref=send_buf.at[s],
                    dst_ref=recv2_buf.at[s],
                    send_sem=send_sems.at[s, 1],
                    recv_sem=recv_sems.at[s, 1],
                    device_id=(left,),
                    device_id_type=pl.DeviceIdType.MESH,
                )
                rr.start()
                rl.start()
                rr.wait()
                rl.wait()
            out_ref[...] = (recv_buf[N_DEV - 2].astype(jnp.float32)
                            + recv2_buf[N_DEV - 2].astype(jnp.float32))
            return

        if _MODE == "split":
            h = m_per // 2
            for s in range(N_DEV - 1):
                rs = []
                for j in range(2):
                    r = pltpu.make_async_remote_copy(
                        src_ref=send_buf.at[s, pl.ds(j * h, h)],
                        dst_ref=recv_buf.at[s, pl.ds(j * h, h)],
                        send_sem=send_sems.at[s, j],
                        recv_sem=recv_sems.at[s, j],
                        device_id=(right,),
                        device_id_type=pl.DeviceIdType.MESH,
                    )
                    r.start()
                    rs.append(r)
                for r in rs:
                    r.wait()
            out_ref[...] = recv_buf[N_DEV - 2].astype(jnp.float32)
            return

        if _MODE == "nocomm":
            acc = partial(my)
            for s in range(N_DEV - 1):
                c = lax.rem(my - (s + 1) + N_DEV, N_DEV)
                send_buf[s] = partial(c).astype(jnp.bfloat16)
                acc = acc + send_buf[s].astype(jnp.float32)
            out_ref[...] = acc * (sx_ref[0] * sw_ref[0])
            return

        rdmas = []
        for s in range(N_DEV - 1):
            c = lax.rem(my - (s + 1) + N_DEV, N_DEV)
            val = partial(c)
            if s > 0:
                rdmas[s - 1].wait_recv()
                val = val + recv_buf[s - 1].astype(jnp.float32)
            send_buf[s] = val.astype(jnp.bfloat16)
            rdma = pltpu.make_async_remote_copy(
                src_ref=send_buf.at[s],
                dst_ref=recv_buf.at[s],
                send_sem=send_sems.at[s, 0],
                recv_sem=recv_sems.at[s, 0],
                device_id=(right,),
                device_id_type=pl.DeviceIdType.MESH,
            )
            rdma.start()
            rdmas.append(rdma)

        p_own = partial(my)
        rdmas[N_DEV - 2].wait_recv()
        scale = sx_ref[0] * sw_ref[0]
        out_ref[...] = (
            p_own + recv_buf[N_DEV - 2].astype(jnp.float32)
        ) * scale
        for r in rdmas:
            r.wait_send()

    return pl.pallas_call(
        body,
        out_shape=jax.ShapeDtypeStruct((m_per, n), jnp.float32),
        in_specs=[
            pl.BlockSpec(memory_space=pltpu.VMEM),
            pl.BlockSpec(memory_space=pltpu.VMEM),
            pl.BlockSpec(memory_space=pltpu.SMEM),
            pl.BlockSpec(memory_space=pltpu.SMEM),
        ],
        out_specs=pl.BlockSpec(memory_space=pltpu.VMEM),
        scratch_shapes=[
            pltpu.VMEM((N_DEV - 1, m_per, n), jnp.bfloat16),
            pltpu.VMEM((N_DEV - 1, m_per, n), jnp.bfloat16),
            pltpu.VMEM((N_DEV - 1, m_per, n), jnp.bfloat16),
            pltpu.SemaphoreType.DMA((N_DEV - 1, 2)),
            pltpu.SemaphoreType.DMA((N_DEV - 1, 2)),
        ],
        compiler_params=pltpu.CompilerParams(collective_id=0),
    )(x, w_mat, scale_x, scale_w)


# device time: 89818 ns/iter; 1.9603x vs baseline; 1.0007x over previous
import os

import jax
import jax.numpy as jnp
from jax import lax
from jax.experimental import pallas as pl
from jax.experimental.pallas import tpu as pltpu

N_DEV = 4
try:
    _MODE = open(os.path.join(os.path.dirname(__file__),
                              "kernel_mode.txt")).read().strip()
except OSError:
    _MODE = "full"


def kernel(x, w_mat, scale_x, scale_w):
    m_global, k_shard = x.shape
    _, n = w_mat.shape
    m_per = m_global // N_DEV

    x = x.astype(jnp.bfloat16)
    w_mat = w_mat.astype(jnp.bfloat16)

    def body(x_ref, w_ref, sx_ref, sw_ref, out_ref,
             send_buf, recv_buf, send_sems, recv_sems):
        my = lax.axis_index("i")
        left = lax.rem(my + N_DEV - 1, N_DEV)
        right = lax.rem(my + 1, N_DEV)

        barrier_sem = pltpu.get_barrier_semaphore()
        for nbr in (left, right):
            pl.semaphore_signal(
                barrier_sem, inc=1,
                device_id=(nbr,), device_id_type=pl.DeviceIdType.MESH,
            )
        pl.semaphore_wait(barrier_sem, 2)

        def partial(c):
            if _MODE == "nocompute":
                return jnp.zeros((m_per, n), jnp.float32)
            xs = x_ref[pl.ds(c * m_per, m_per), :]
            return jnp.dot(xs, w_ref[...], preferred_element_type=jnp.float32)

        if _MODE == "half":
            h = m_per // 2
            for s in range(N_DEV - 1):
                r = pltpu.make_async_remote_copy(
                    src_ref=send_buf.at[s, pl.ds(0, h)],
                    dst_ref=recv_buf.at[s, pl.ds(0, h)],
                    send_sem=send_sems.at[s, 0],
                    recv_sem=recv_sems.at[s, 0],
                    device_id=(right,),
                    device_id_type=pl.DeviceIdType.MESH,
                )
                r.start()
                r.wait()
            out_ref[...] = recv_buf[N_DEV - 2].astype(jnp.float32)
            return

        if _MODE == "bidir":
            h = m_per // 2
            for s in range(N_DEV - 1):
                rr = pltpu.make_async_remote_copy(
                    src_ref=send_buf.at[s, pl.ds(0, h)],
                    dst_ref=recv_buf.at[s, pl.ds(0, h)],
                    send_sem=send_sems.at[s, 0],
                    recv_sem=recv_sems.at[s, 0],
                    device_id=(right,),
                    device_id_type=pl.DeviceIdType.MESH,
                )
                rl = pltpu.make_async_remote_copy(
                    src_ref=send_buf.at[s, pl.ds(h, h)],
                    dst_ref=recv_buf.at[s, pl.ds(h, h)],
                    send_sem=send_sems.at[s, 1],
                    recv_sem=recv_sems.at[s, 1],
                    device_id=(left,),
                    device_id_type=pl.DeviceIdType.MESH,
                )
                rr.start()
                rl.start()
                rr.wait()
                rl.wait()
            out_ref[...] = recv_buf[N_DEV - 2].astype(jnp.float32)
            return

        if _MODE == "split":
            h = m_per // 2
            for s in range(N_DEV - 1):
                rs = []
                for j in range(2):
                    r = pltpu.make_async_remote_copy(
                        src_ref=send_buf.at[s, pl.ds(j * h, h)],
                        dst_ref=recv_buf.at[s, pl.ds(j * h, h)],
                        send_sem=send_sems.at[s, j],
                        recv_sem=recv_sems.at[s, j],
                        device_id=(right,),
                        device_id_type=pl.DeviceIdType.MESH,
                    )
                    r.start()
                    rs.append(r)
                for r in rs:
                    r.wait()
            out_ref[...] = recv_buf[N_DEV - 2].astype(jnp.float32)
            return

        if _MODE == "nocomm":
            acc = partial(my)
            for s in range(N_DEV - 1):
                c = lax.rem(my - (s + 1) + N_DEV, N_DEV)
                send_buf[s] = partial(c).astype(jnp.bfloat16)
                acc = acc + send_buf[s].astype(jnp.float32)
            out_ref[...] = acc * (sx_ref[0] * sw_ref[0])
            return

        rdmas = []
        for s in range(N_DEV - 1):
            c = lax.rem(my - (s + 1) + N_DEV, N_DEV)
            val = partial(c)
            if s > 0:
                rdmas[s - 1].wait_recv()
                val = val + recv_buf[s - 1].astype(jnp.float32)
            send_buf[s] = val.astype(jnp.bfloat16)
            rdma = pltpu.make_async_remote_copy(
                src_ref=send_buf.at[s],
                dst_ref=recv_buf.at[s],
                send_sem=send_sems.at[s, 0],
                recv_sem=recv_sems.at[s, 0],
                device_id=(right,),
                device_id_type=pl.DeviceIdType.MESH,
            )
            rdma.start()
            rdmas.append(rdma)

        p_own = partial(my)
        rdmas[N_DEV - 2].wait_recv()
        scale = sx_ref[0] * sw_ref[0]
        out_ref[...] = (
            p_own + recv_buf[N_DEV - 2].astype(jnp.float32)
        ) * scale
        for r in rdmas:
            r.wait_send()

    return pl.pallas_call(
        body,
        out_shape=jax.ShapeDtypeStruct((m_per, n), jnp.float32),
        in_specs=[
            pl.BlockSpec(memory_space=pltpu.VMEM),
            pl.BlockSpec(memory_space=pltpu.VMEM),
            pl.BlockSpec(memory_space=pltpu.SMEM),
            pl.BlockSpec(memory_space=pltpu.SMEM),
        ],
        out_specs=pl.BlockSpec(memory_space=pltpu.VMEM),
        scratch_shapes=[
            pltpu.VMEM((N_DEV - 1, m_per, n), jnp.bfloat16),
            pltpu.VMEM((N_DEV - 1, m_per, n), jnp.bfloat16),
            pltpu.SemaphoreType.DMA((N_DEV - 1, 2)),
            pltpu.SemaphoreType.DMA((N_DEV - 1, 2)),
        ],
        compiler_params=pltpu.CompilerParams(collective_id=0),
    )(x, w_mat, scale_x, scale_w)
